# Initial kernel scaffold; baseline (speedup 1.0000x reference)
import jax
import jax.numpy as jnp
from jax.experimental import pallas as pl


def kernel(x, router_logits, w1, w3, w2):
    raise NotImplementedError("write your pallas kernel here")


# speedup vs baseline: 2.7956x; 2.7956x over previous
import functools

import jax
import jax.numpy as jnp
from jax.experimental import pallas as pl
from jax.experimental.pallas import tpu as pltpu

E = 64
T = 128
D = 1024
F = 1024


def _moe_kernel(x_ref, logits_ref, w1_ref, w3_ref, w2_ref, out_ref,
                i1_ref, i2_ref, s1_ref, s2_ref):
    e = pl.program_id(0)

    @pl.when(e == 0)
    def _routing_prologue():
        logits = logits_ref[...]
        eids = jax.lax.broadcasted_iota(jnp.int32, (T, E), 1)
        l1 = jnp.max(logits, axis=1, keepdims=True)
        i1 = jnp.min(jnp.where(logits == l1, eids, E), axis=1, keepdims=True)
        masked = jnp.where(eids == i1, -jnp.inf, logits)
        l2 = jnp.max(masked, axis=1, keepdims=True)
        i2 = jnp.min(jnp.where(masked == l2, eids, E), axis=1, keepdims=True)
        s1 = 1.0 / (1.0 + jnp.exp(l2 - l1))
        i1_ref[...] = i1
        i2_ref[...] = i2
        s1_ref[...] = s1
        s2_ref[...] = 1.0 - s1

    xb = x_ref[...]
    g = jax.lax.dot_general(
        xb, w1_ref[0], (((1,), (1,)), ((), ())),
        preferred_element_type=jnp.float32,
    )
    u = jax.lax.dot_general(
        xb, w3_ref[0], (((1,), (1,)), ((), ())),
        preferred_element_type=jnp.float32,
    )
    h = (g * jax.nn.sigmoid(g)) * u
    scale = (jnp.where(i1_ref[...] == e, s1_ref[...], 0.0)
             + jnp.where(i2_ref[...] == e, s2_ref[...], 0.0))
    y = jax.lax.dot_general(
        h * scale, w2_ref[0], (((1,), (1,)), ((), ())),
        preferred_element_type=jnp.float32,
    )

    @pl.when(e == 0)
    def _init():
        out_ref[...] = y

    @pl.when(e > 0)
    def _acc():
        out_ref[...] += y


@jax.jit
def kernel(x, router_logits, w1, w3, w2):
    return pl.pallas_call(
        _moe_kernel,
        grid=(E,),
        in_specs=[
            pl.BlockSpec((T, D), lambda e: (0, 0)),
            pl.BlockSpec((T, E), lambda e: (0, 0)),
            pl.BlockSpec((1, F, D), lambda e: (e, 0, 0)),
            pl.BlockSpec((1, F, D), lambda e: (e, 0, 0)),
            pl.BlockSpec((1, D, F), lambda e: (e, 0, 0)),
        ],
        out_specs=pl.BlockSpec((T, D), lambda e: (0, 0)),
        out_shape=jax.ShapeDtypeStruct((T, D), jnp.float32),
        scratch_shapes=[
            pltpu.VMEM((T, 1), jnp.int32),
            pltpu.VMEM((T, 1), jnp.int32),
            pltpu.VMEM((T, 1), jnp.float32),
            pltpu.VMEM((T, 1), jnp.float32),
        ],
    )(x, router_logits, w1, w3, w2)
